# Initial kernel scaffold; baseline (speedup 1.0000x reference)
import jax
import jax.numpy as jnp
from jax.experimental import pallas as pl


def kernel(peptide_indices, atom_indices, residue_names, atom_names, W_res, W_atom):
    raise NotImplementedError("write your pallas kernel here")


# speedup vs baseline: 2.0654x; 2.0654x over previous
import functools

import jax
import jax.numpy as jnp
from jax.experimental import pallas as pl
from jax.experimental.pallas import tpu as pltpu

AA_DIM = 128
MAX_ATOM_INDX = 14.0
RES_VOCAB = 26
RES_DIM = 128
ATOM_VOCAB = 128
ATOM_DIM = 64
OUT_DIM = 1 + AA_DIM + RES_DIM + ATOM_DIM

ROWS = 1024


def _body(pep_ref, atom_ref, res_ref, an_ref, wres_ref, watom_ref, out_ref):
    pep = pep_ref[0]
    atom_idx = atom_ref[0]
    res_ids = res_ref[0]
    atom_ids = an_ref[0]

    half = AA_DIM // 2
    scale = jnp.log(MAX_ATOM_INDX) / (half - 1)
    freqs = jnp.exp(
        jax.lax.broadcasted_iota(jnp.int32, (1, half), 1).astype(jnp.float32)
        * (-scale)
    )
    pe = atom_idx * freqs

    res_onehot = (
        res_ids == jax.lax.broadcasted_iota(jnp.int32, (1, RES_VOCAB), 1)
    ).astype(jnp.float32)
    res_emb = jax.lax.dot(
        res_onehot, wres_ref[...], preferred_element_type=jnp.float32
    )
    atom_onehot = (
        atom_ids == jax.lax.broadcasted_iota(jnp.int32, (1, ATOM_VOCAB), 1)
    ).astype(jnp.float32)
    atom_emb = jax.lax.dot(
        atom_onehot, watom_ref[...], preferred_element_type=jnp.float32
    )

    out_ref[:, 0:1] = pep
    out_ref[:, 1 : 1 + half] = jnp.sin(pe)
    out_ref[:, 1 + half : 1 + AA_DIM] = jnp.cos(pe)
    out_ref[:, 1 + AA_DIM : 1 + AA_DIM + RES_DIM] = res_emb
    out_ref[:, 1 + AA_DIM + RES_DIM :] = atom_emb


@jax.jit
def kernel(peptide_indices, atom_indices, residue_names, atom_names, W_res, W_atom):
    B, L = peptide_indices.shape
    n = B * L
    nb = n // ROWS

    def col(x):
        return x.reshape(nb, ROWS, 1)

    row_spec = pl.BlockSpec((1, ROWS, 1), lambda i: (i, 0, 0))
    out2d = pl.pallas_call(
        _body,
        grid=(nb,),
        in_specs=[
            row_spec,
            row_spec,
            row_spec,
            row_spec,
            pl.BlockSpec((RES_VOCAB, RES_DIM), lambda i: (0, 0)),
            pl.BlockSpec((ATOM_VOCAB, ATOM_DIM), lambda i: (0, 0)),
        ],
        out_specs=pl.BlockSpec((ROWS, OUT_DIM), lambda i: (i, 0)),
        out_shape=jax.ShapeDtypeStruct((n, OUT_DIM), jnp.float32),
        compiler_params=pltpu.CompilerParams(
            dimension_semantics=("parallel",),
        ),
    )(
        col(peptide_indices),
        col(atom_indices),
        col(residue_names),
        col(atom_names),
        W_res,
        W_atom,
    )
    return out2d.reshape(B, L, OUT_DIM)


# speedup vs baseline: 2.4332x; 1.1781x over previous
import functools

import jax
import jax.numpy as jnp
from jax.experimental import pallas as pl
from jax.experimental.pallas import tpu as pltpu

AA_DIM = 128
MAX_ATOM_INDX = 14.0
RES_VOCAB = 26
RES_DIM = 128
ATOM_VOCAB = 128
ATOM_DIM = 64
OUT_DIM = 1 + AA_DIM + RES_DIM + ATOM_DIM

ROWS = 1024

_PI2_HI = 1.5707963705062866
_PI2_LO = -4.371139000186241e-08
_INV_PI2 = 0.6366197723675814


def _sincos(x):
    k = jnp.round(x * _INV_PI2)
    r = (x - k * _PI2_HI) - k * _PI2_LO
    r2 = r * r
    sp = r + r * r2 * (-1.6666654611e-1 + r2 * (8.3321608736e-3 + r2 * (-1.9515295891e-4)))
    cp = 1.0 + r2 * (-0.5 + r2 * (4.166664568298827e-2 + r2 * (-1.388731625493765e-3)))
    q = k.astype(jnp.int32)
    odd = (q & 1) == 1
    sin_mag = jnp.where(odd, cp, sp)
    cos_mag = jnp.where(odd, sp, cp)
    qm = q & 3
    sin_neg = qm >= 2
    cos_neg = (qm == 1) | (qm == 2)
    s = jnp.where(sin_neg, -sin_mag, sin_mag)
    c = jnp.where(cos_neg, -cos_mag, cos_mag)
    return s, c


def _body(pep_ref, atom_ref, res_ref, an_ref, wres_ref, watom_ref, out_ref):
    pep = pep_ref[0]
    atom_idx = atom_ref[0]
    res_ids = res_ref[0]
    atom_ids = an_ref[0]

    half = AA_DIM // 2
    scale = jnp.log(MAX_ATOM_INDX) / (half - 1)
    freqs = jnp.exp(
        jax.lax.broadcasted_iota(jnp.int32, (1, half), 1).astype(jnp.float32)
        * (-scale)
    )
    pe = atom_idx * freqs

    res_onehot = (
        res_ids == jax.lax.broadcasted_iota(jnp.int32, (1, RES_VOCAB), 1)
    ).astype(jnp.float32)
    res_emb = jax.lax.dot(
        res_onehot, wres_ref[...], preferred_element_type=jnp.float32
    )
    atom_onehot = (
        atom_ids == jax.lax.broadcasted_iota(jnp.int32, (1, ATOM_VOCAB), 1)
    ).astype(jnp.float32)
    atom_emb = jax.lax.dot(
        atom_onehot, watom_ref[...], preferred_element_type=jnp.float32
    )

    s, c = _sincos(pe)
    out_ref[:, 0:1] = pep
    out_ref[:, 1 : 1 + half] = s
    out_ref[:, 1 + half : 1 + AA_DIM] = c
    out_ref[:, 1 + AA_DIM : 1 + AA_DIM + RES_DIM] = res_emb
    out_ref[:, 1 + AA_DIM + RES_DIM :] = atom_emb


@jax.jit
def kernel(peptide_indices, atom_indices, residue_names, atom_names, W_res, W_atom):
    B, L = peptide_indices.shape
    n = B * L
    nb = n // ROWS

    def col(x):
        return x.reshape(nb, ROWS, 1)

    row_spec = pl.BlockSpec((1, ROWS, 1), lambda i: (i, 0, 0))
    out2d = pl.pallas_call(
        _body,
        grid=(nb,),
        in_specs=[
            row_spec,
            row_spec,
            row_spec,
            row_spec,
            pl.BlockSpec((RES_VOCAB, RES_DIM), lambda i: (0, 0)),
            pl.BlockSpec((ATOM_VOCAB, ATOM_DIM), lambda i: (0, 0)),
        ],
        out_specs=pl.BlockSpec((ROWS, OUT_DIM), lambda i: (i, 0)),
        out_shape=jax.ShapeDtypeStruct((n, OUT_DIM), jnp.float32),
        compiler_params=pltpu.CompilerParams(
            dimension_semantics=("parallel",),
        ),
    )(
        col(peptide_indices),
        col(atom_indices),
        col(residue_names),
        col(atom_names),
        W_res,
        W_atom,
    )
    return out2d.reshape(B, L, OUT_DIM)
